# baseline (speedup 1.0000x reference)
import functools

import jax
import jax.numpy as jnp
from jax import lax
from jax.experimental import pallas as pl
from jax.experimental.pallas import tpu as pltpu
from jax.experimental.pallas import tpu_sc as plsc

BATCH = 16384
D_ID = 128
D_REVIEW = 64
CHUNK = 128


def _build_kernels():
    info = plsc.get_sparse_core_info()
    num_cores = info.num_cores
    num_workers = num_cores * info.num_subcores
    b_per_w = BATCH // num_workers
    n_chunks = b_per_w // CHUNK

    mesh = plsc.VectorSubcoreMesh(core_axis_name="c", subcore_axis_name="s")
    params = pltpu.CompilerParams(
        use_tc_tiling_on_sc=True, needs_layout_passes=False)

    def make_gather(n_tables):
        @functools.partial(
            pl.kernel,
            mesh=mesh,
            compiler_params=params,
            out_type=[jax.ShapeDtypeStruct((BATCH, D_ID), jnp.float32)
                      for _ in range(n_tables)],
            scratch_types=(
                [pltpu.VMEM((n_chunks, CHUNK), jnp.int32)]
                + [pltpu.VMEM((CHUNK, D_ID), jnp.float32)
                   for _ in range(2 * n_tables)]
                + [pltpu.SemaphoreType.DMA] * 4
            ),
        )
        def gather_k(*refs):
            idx_hbm = refs[0]
            tables = refs[1:1 + n_tables]
            outs = refs[1 + n_tables:1 + 2 * n_tables]
            idx_v = refs[1 + 2 * n_tables]
            vbufs = refs[2 + 2 * n_tables:2 + 2 * n_tables + 2 * n_tables]
            gs0, gs1, os0, os1 = refs[-4:]

            wid = lax.axis_index("s") * num_cores + lax.axis_index("c")
            base = wid * b_per_w
            for j in range(n_chunks):
                pltpu.sync_copy(idx_hbm.at[pl.ds(base + j * CHUNK, CHUNK)],
                                idx_v.at[j])

            slots = (
                (vbufs[:n_tables], gs0, os0),
                (vbufs[n_tables:], gs1, os1),
            )

            def start_gather(j, s):
                bufs, gs, _ = slots[s]
                return [pltpu.async_copy(t.at[idx_v.at[j]], b, gs)
                        for t, b in zip(tables, bufs)]

            def start_copyout(j, s):
                bufs, _, os = slots[s]
                off = base + j * CHUNK
                return [pltpu.async_copy(b, o.at[pl.ds(off, CHUNK)], os)
                        for b, o in zip(bufs, outs)]

            gather_h = [None, None]
            copy_h = [None, None]
            for j in range(min(2, n_chunks)):
                gather_h[j] = start_gather(j, j)
            for j in range(n_chunks):
                s = j % 2
                if copy_h[s] is not None:
                    for h in copy_h[s]:
                        h.wait()
                    copy_h[s] = None
                    gather_h[s] = start_gather(j, s)
                for h in gather_h[s]:
                    h.wait()
                copy_h[s] = start_copyout(j, s)
            for s in range(2):
                if copy_h[s] is not None:
                    for h in copy_h[s]:
                        h.wait()

        return gather_k

    return make_gather(2), make_gather(1)


def kernel(nodes_u, global_protos, u_emb_weight, u_review_weight):
    kernel_pe, kernel_rev = _build_kernels()
    idx = nodes_u.astype(jnp.int32)
    rev_pad = jnp.pad(u_review_weight, ((0, 0), (0, D_ID - D_REVIEW)))
    proto_feats, u_id_feats = kernel_pe(idx, global_protos, u_emb_weight)
    (review_wide,) = kernel_rev(idx, rev_pad)
    return (proto_feats, u_id_feats, review_wide[:, :D_REVIEW])


# speedup vs baseline: 1.0114x; 1.0114x over previous
import functools

import jax
import jax.numpy as jnp
from jax import lax
from jax.experimental import pallas as pl
from jax.experimental.pallas import tpu as pltpu
from jax.experimental.pallas import tpu_sc as plsc

BATCH = 16384
D_ID = 128
D_REVIEW = 64
CHUNK = 128


def _build_kernels():
    info = plsc.get_sparse_core_info()
    num_cores = info.num_cores
    num_workers = num_cores * info.num_subcores
    b_per_w = BATCH // num_workers
    n_chunks = b_per_w // CHUNK

    mesh = plsc.VectorSubcoreMesh(core_axis_name="c", subcore_axis_name="s")
    params = pltpu.CompilerParams(
        use_tc_tiling_on_sc=True, needs_layout_passes=False)

    def make_gather(n_tables):
        @functools.partial(
            pl.kernel,
            mesh=mesh,
            compiler_params=params,
            out_type=[jax.ShapeDtypeStruct((BATCH, D_ID), jnp.float32)
                      for _ in range(n_tables)],
            scratch_types=(
                [pltpu.VMEM((n_chunks, CHUNK), jnp.int32)]
                + [pltpu.VMEM((CHUNK, D_ID), jnp.float32)
                   for _ in range(2 * n_tables)]
                + [pltpu.SemaphoreType.DMA] * 4
            ),
        )
        def gather_k(*refs):
            idx_hbm = refs[0]
            tables = refs[1:1 + n_tables]
            outs = refs[1 + n_tables:1 + 2 * n_tables]
            idx_v = refs[1 + 2 * n_tables]
            vbufs = refs[2 + 2 * n_tables:2 + 2 * n_tables + 2 * n_tables]
            gs0, gs1, os0, os1 = refs[-4:]

            wid = lax.axis_index("s") * num_cores + lax.axis_index("c")
            base = wid * b_per_w
            for j in range(n_chunks):
                pltpu.sync_copy(idx_hbm.at[pl.ds(base + j * CHUNK, CHUNK)],
                                idx_v.at[j])

            slots = (
                (vbufs[:n_tables], gs0, os0),
                (vbufs[n_tables:], gs1, os1),
            )

            def start_gather(j, s):
                bufs, gs, _ = slots[s]
                return [pltpu.async_copy(t.at[idx_v.at[j]], b, gs)
                        for t, b in zip(tables, bufs)]

            def start_copyout(j, s):
                bufs, _, os = slots[s]
                off = base + j * CHUNK
                return [pltpu.async_copy(b, o.at[pl.ds(off, CHUNK)], os)
                        for b, o in zip(bufs, outs)]

            gather_h = [None, None]
            copy_h = [None, None]
            for j in range(min(2, n_chunks)):
                gather_h[j] = start_gather(j, j)
            for j in range(n_chunks):
                s = j % 2
                if copy_h[s] is not None:
                    for h in copy_h[s]:
                        h.wait()
                    copy_h[s] = None
                    gather_h[s] = start_gather(j, s)
                for h in gather_h[s]:
                    h.wait()
                copy_h[s] = start_copyout(j, s)
            for s in range(2):
                if copy_h[s] is not None:
                    for h in copy_h[s]:
                        h.wait()

        return gather_k

    return make_gather(2), make_gather(1)


def kernel(nodes_u, global_protos, u_emb_weight, u_review_weight):
    kernel_pe, kernel_rev = _build_kernels()
    idx = nodes_u.astype(jnp.int32)
    rev_pad = jnp.pad(u_review_weight, ((0, 0), (0, D_ID - D_REVIEW)))
    proto_feats, u_id_feats = kernel_pe(idx, global_protos, u_emb_weight)
    rev_pad, proto_feats, u_id_feats = lax.optimization_barrier(
        (rev_pad, proto_feats, u_id_feats))
    (review_wide,) = kernel_rev(idx, rev_pad)
    return (proto_feats, u_id_feats, review_wide[:, :D_REVIEW])
